# Initial kernel scaffold; baseline (speedup 1.0000x reference)
import functools

import jax
import jax.numpy as jnp
from jax import lax
from jax.experimental import pallas as pl
from jax.experimental.pallas import tpu as pltpu
import jax.experimental.pallas.tpu_sc as plsc

N = 50000
HID = 20
TW = 32
NC, NS, L = 2, 16, 16
NW = NC * NS
KB = 128
SB = 8
K = KB * SB
R = 3126
NP = R * NS
RPT = NP // NS
NEG = -1.0e30


def _lrelu(x):
    return jnp.where(x >= 0, x, 0.2 * x)


def _a_body(x_ref, w_ref, a2_ref, table_ref, ed_ref, gmax_ref):
    i = pl.program_id(0)
    h = jnp.dot(x_ref[...], w_ref[...], preferred_element_type=jnp.float32)
    esed = jnp.dot(h, a2_ref[...], preferred_element_type=jnp.float32)
    rows = i * R + lax.broadcasted_iota(jnp.int32, (R, 1), 0)
    valid = rows < N
    pad = jnp.zeros((R, TW - HID - 2), jnp.float32)
    table_ref[...] = jnp.concatenate([h, esed, pad], axis=1)
    ed_ref[...] = jnp.where(valid, esed[:, 1:2], 0.0)
    es_m = jnp.where(valid, esed[:, 0:1], NEG)
    bm = jnp.max(es_m)

    @pl.when(i == 0)
    def _():
        gmax_ref[...] = jnp.full((1, 8), NEG, jnp.float32)

    gmax_ref[...] = jnp.maximum(gmax_ref[...], bm)


def _dense0(features, w0, a2):
    return pl.pallas_call(
        _a_body,
        grid=(NS,),
        in_specs=[
            pl.BlockSpec((R, features.shape[1]), lambda i: (i, 0)),
            pl.BlockSpec(w0.shape, lambda i: (0, 0)),
            pl.BlockSpec(a2.shape, lambda i: (0, 0)),
        ],
        out_specs=[
            pl.BlockSpec((R, TW), lambda i: (i, 0)),
            pl.BlockSpec((R, 1), lambda i: (i, 0)),
            pl.BlockSpec((1, 8), lambda i: (0, 0)),
        ],
        out_shape=[
            jax.ShapeDtypeStruct((N, TW), jnp.float32),
            jax.ShapeDtypeStruct((NP, 1), jnp.float32),
            jax.ShapeDtypeStruct((1, 8), jnp.float32),
        ],
    )(features, w0, a2)


def _edge_body(cpw, src_hbm, dst_hbm, table_hbm, ed_hbm, gmax_hbm, out_hbm,
               src2, dst2, rows, edv, gv, fbuf, acc, gsem, ssem):
    cid = lax.axis_index("c")
    sid = lax.axis_index("s")
    w = sid * NC + cid

    pltpu.sync_copy(ed_hbm, edv)
    pltpu.sync_copy(gmax_hbm, gv)

    zv = jnp.zeros((L,), jnp.float32)

    def _zero_rows(i, _):
        rows[i, pl.ds(0, L)] = zv
        rows[i, pl.ds(L, L)] = zv
        return 0

    lax.fori_loop(0, K, _zero_rows, 0)
    row0 = sid * RPT
    nfull = RPT // K
    rem = RPT - nfull * K
    for q in range(nfull):
        pltpu.sync_copy(rows, acc.at[pl.ds(row0 + q * K, K)])
    if rem:
        pltpu.sync_copy(rows.at[pl.ds(0, rem)],
                        acc.at[pl.ds(row0 + nfull * K, rem)])
    plsc.subcore_barrier()

    g = gv[0, 0]
    lane = lax.broadcasted_iota(jnp.int32, (L,), 0)

    def _chunk(t, _):
        ci = w * cpw + t
        pltpu.sync_copy(src_hbm.at[ci], src2)
        pltpu.sync_copy(dst_hbm.at[ci], dst2)
        descs = []
        for j in range(SB):
            descs.append(pltpu.async_copy(
                table_hbm.at[src2.at[j]], rows.at[pl.ds(j * KB, KB)], gsem))
        for dsc in descs:
            dsc.wait()

        for j in range(SB):
            def _group(q, _, j=j):
                k0 = q * L
                dst_g = dst2[j, pl.ds(k0, L)]
                ed_g = plsc.load_gather(edv, [dst_g])
                kv = (j * KB + k0) + lane
                es_g = plsc.load_gather(rows, [kv, jnp.full((L,), HID, jnp.int32)])
                e = _lrelu(es_g + ed_g)
                c = _lrelu(g + ed_g)
                f = jnp.exp(e - c)
                fbuf[...] = f
                for i in range(L):
                    kk = j * KB + k0 + i
                    fs = fbuf[i]
                    r0 = rows[kk, pl.ds(0, L)]
                    r1 = rows[kk, pl.ds(L, L)]
                    rows[kk, pl.ds(0, L)] = r0 * fs
                    rows[kk, pl.ds(L, L)] = jnp.where(lane == (HID - L), fs,
                                                      r1 * fs)
                return 0
            lax.fori_loop(0, KB // L, _group, 0)

        sdescs = []
        for j in range(SB):
            sdescs.append(pltpu.async_copy(
                rows.at[pl.ds(j * KB, KB)], acc.at[dst2.at[j]], ssem,
                add=True))
        for dsc in sdescs:
            dsc.wait()
        return 0

    lax.fori_loop(0, cpw, _chunk, 0)

    plsc.subcore_barrier()
    pltpu.sync_copy(acc.at[pl.ds(row0, RPT)],
                    out_hbm.at[cid, pl.ds(row0, RPT)])


def _edge_pass(src3, dst3, table, ed, gmax):
    cpw = src3.shape[0] // NW
    body = functools.partial(_edge_body, cpw)
    return pl.kernel(
        body,
        out_type=jax.ShapeDtypeStruct((NC, NP, TW), jnp.float32),
        mesh=plsc.VectorSubcoreMesh(core_axis_name="c", subcore_axis_name="s",
                                    num_cores=NC, num_subcores=NS),
        scratch_types=[
            pltpu.VMEM((SB, KB), jnp.int32),
            pltpu.VMEM((SB, KB), jnp.int32),
            pltpu.VMEM((K, TW), jnp.float32),
            pltpu.VMEM((NP,), jnp.float32),
            pltpu.VMEM((1, 8), jnp.float32),
            pltpu.VMEM((L,), jnp.float32),
            pltpu.VMEM_SHARED((NP, TW), jnp.float32),
            pltpu.SemaphoreType.DMA,
            pltpu.SemaphoreType.DMA,
        ],
    )(src3, dst3, table, ed, gmax)


def _c_body(l, accp_ref, prev_ref, emb_ref, gam_ref, bet_ref, mu_ref, var_ref,
            b_ref, lw_ref, wn_ref, a2_ref, ow_ref, ob_ref,
            out_ref, embo_ref, table_ref, ed_ref, gmax_ref, fin_ref):
    i = pl.program_id(0)
    s = accp_ref[0] + accp_ref[1]
    agg = s[:, 0:HID]
    den = s[:, HID:HID + 1]
    y = agg / (den + 1e-16)
    scale = gam_ref[...] * lax.rsqrt(var_ref[...] + 1e-5)
    y = (y + b_ref[...] - mu_ref[...]) * scale + bet_ref[...]
    y = jnp.maximum(y, 0.0)
    if l > 0:
        y = y + 0.7 * prev_ref[...]
    out_ref[...] = y
    lw = lw_ref[...]
    wsm = jax.nn.softmax(lw, axis=1)
    emb = wsm[0, l] * y
    if l > 0:
        emb = emb + emb_ref[...]
    if l < 3:
        embo_ref[...] = emb
        h = jnp.dot(y, wn_ref[...], preferred_element_type=jnp.float32)
        esed = jnp.dot(h, a2_ref[...], preferred_element_type=jnp.float32)
        rows = i * R + lax.broadcasted_iota(jnp.int32, (R, 1), 0)
        valid = rows < N
        pad = jnp.zeros((R, TW - HID - 2), jnp.float32)
        table_ref[...] = jnp.concatenate([h, esed, pad], axis=1)
        ed_ref[...] = jnp.where(valid, esed[:, 1:2], 0.0)
        es_m = jnp.where(valid, esed[:, 0:1], NEG)
        bm = jnp.max(es_m)

        @pl.when(i == 0)
        def _():
            gmax_ref[...] = jnp.full((1, 8), NEG, jnp.float32)

        gmax_ref[...] = jnp.maximum(gmax_ref[...], bm)
    else:
        fin_ref[...] = jnp.dot(emb, ow_ref[...],
                               preferred_element_type=jnp.float32) + ob_ref[...]


def _post(l, accp, prev, emb, bnp, bvec, lw, wn, a2, ow, ob):
    v20 = lambda a: a.reshape(1, HID)
    body = functools.partial(_c_body, l)
    small = lambda s: pl.BlockSpec(s, lambda i: (0, 0))
    outs = [
        jax.ShapeDtypeStruct((N, HID), jnp.float32),
        jax.ShapeDtypeStruct((N, HID), jnp.float32),
        jax.ShapeDtypeStruct((N, TW), jnp.float32),
        jax.ShapeDtypeStruct((NP, 1), jnp.float32),
        jax.ShapeDtypeStruct((1, 8), jnp.float32),
        jax.ShapeDtypeStruct((N, 2), jnp.float32),
    ]
    out_specs = [
        pl.BlockSpec((R, HID), lambda i: (i, 0)),
        pl.BlockSpec((R, HID), lambda i: (i, 0)),
        pl.BlockSpec((R, TW), lambda i: (i, 0)),
        pl.BlockSpec((R, 1), lambda i: (i, 0)),
        pl.BlockSpec((1, 8), lambda i: (0, 0)),
        pl.BlockSpec((R, 2), lambda i: (i, 0)),
    ]
    return pl.pallas_call(
        body,
        grid=(NS,),
        in_specs=[
            pl.BlockSpec((2, R, TW), lambda i: (0, i, 0)),
            pl.BlockSpec((R, HID), lambda i: (i, 0)),
            pl.BlockSpec((R, HID), lambda i: (i, 0)),
            small((1, HID)), small((1, HID)), small((1, HID)),
            small((1, HID)), small((1, HID)), small((1, 4)),
            small((HID, HID)), small((HID, 2)), small((HID, 2)),
            small((1, 2)),
        ],
        out_specs=out_specs,
        out_shape=outs,
    )(accp, prev, emb,
      v20(bnp['gamma']), v20(bnp['beta']), v20(bnp['mean']), v20(bnp['var']),
      bvec.reshape(1, HID), lw.reshape(1, 4), wn, a2, ow, ob.reshape(1, 2))


def kernel(features, edges, edges_weight, params):
    del edges_weight
    gat = params['gat']
    bn = params['bn']

    e_real = edges.shape[1] + N
    ct = -(-e_real // K)
    ct = -(-ct // NW) * NW
    ep = ct * K
    loop = jnp.arange(N, dtype=jnp.int32)
    srcp = jnp.concatenate(
        [edges[0], loop, jnp.zeros((ep - e_real,), jnp.int32)])
    dstp = jnp.concatenate(
        [edges[1], loop, jnp.full((ep - e_real,), N, jnp.int32)])
    src3 = srcp.reshape(ct, SB, KB)
    dst3 = dstp.reshape(ct, SB, KB)

    a2 = [jnp.stack([p['a_src'], p['a_dst']], axis=1) for p in gat]

    table, ed, gmax = _dense0(features, gat[0]['W'], a2[0])
    prev = jnp.zeros((N, HID), jnp.float32)
    emb = jnp.zeros((N, HID), jnp.float32)
    fin = None
    for l in range(4):
        accp = _edge_pass(src3, dst3, table, ed.reshape(NP), gmax)
        wn = gat[l + 1]['W'] if l < 3 else jnp.zeros((HID, HID), jnp.float32)
        a2n = a2[l + 1] if l < 3 else jnp.zeros((HID, 2), jnp.float32)
        out_l, emb_n, table, ed, gmax, fin = _post(
            l, accp, prev, emb, bn[l], gat[l]['b'], params['layer_weights'],
            wn, a2n, params['out_W'], params['out_b'])
        prev, emb = out_l, emb_n
    return fin


# speedup vs baseline: 85.3283x; 85.3283x over previous
import functools

import jax
import jax.numpy as jnp
from jax import lax
from jax.experimental import pallas as pl
from jax.experimental.pallas import tpu as pltpu
import jax.experimental.pallas.tpu_sc as plsc

N = 50000
HID = 20
TW = 32
NC, NS, L = 2, 16, 16
NW = NC * NS
KB = 128
SB = 4
K = KB * SB
R = 3128
NP = R * NS
RPT = NP // NS
NEG = -1.0e30


def _lrelu(x):
    return jnp.where(x >= 0, x, 0.2 * x)


def _a_body(x_ref, w_ref, a2_ref, table_ref, ed_ref, gmax_ref):
    i = pl.program_id(0)
    h = jnp.dot(x_ref[...], w_ref[...], preferred_element_type=jnp.float32)
    esed = jnp.dot(h, a2_ref[...], preferred_element_type=jnp.float32)
    rows = i * R + lax.broadcasted_iota(jnp.int32, (R, 1), 0)
    valid = rows < N
    pad = jnp.zeros((R, TW - HID - 2), jnp.float32)
    table_ref[...] = jnp.concatenate([h, esed, pad], axis=1)
    edc = jnp.where(valid, esed[:, 1:2], 0.0)
    ed_ref[...] = jnp.concatenate([edc, jnp.zeros((R, 7), jnp.float32)],
                                  axis=1)
    es_m = jnp.where(valid, esed[:, 0:1], NEG)
    bm = jnp.max(es_m)

    @pl.when(i == 0)
    def _():
        gmax_ref[...] = jnp.full((1, 16), NEG, jnp.float32)

    gmax_ref[...] = jnp.maximum(gmax_ref[...], bm)


def _dense0(features, w0, a2):
    return pl.pallas_call(
        _a_body,
        grid=(NS,),
        in_specs=[
            pl.BlockSpec((R, features.shape[1]), lambda i: (i, 0)),
            pl.BlockSpec(w0.shape, lambda i: (0, 0)),
            pl.BlockSpec(a2.shape, lambda i: (0, 0)),
        ],
        out_specs=[
            pl.BlockSpec((R, TW), lambda i: (i, 0)),
            pl.BlockSpec((R, 8), lambda i: (i, 0)),
            pl.BlockSpec((1, 16), lambda i: (0, 0)),
        ],
        out_shape=[
            jax.ShapeDtypeStruct((N, TW), jnp.float32),
            jax.ShapeDtypeStruct((NP, 8), jnp.float32),
            jax.ShapeDtypeStruct((1, 16), jnp.float32),
        ],
    )(features, w0, a2)


def _edge_body(cpw, src_hbm, dst_hbm, table_hbm, ed_hbm, gmax_hbm, out_hbm,
               srcb, dstb, edb, rows, gv, acc, gsem, ssem):
    cid = lax.axis_index("c")
    sid = lax.axis_index("s")
    w = sid * NC + cid

    pltpu.sync_copy(gmax_hbm, gv)

    zv = jnp.zeros((L,), jnp.float32)

    def _zero_rows(i, _):
        rows[i, pl.ds(0, L)] = zv
        rows[i, pl.ds(L, L)] = zv
        return 0

    lax.fori_loop(0, K, _zero_rows, 0)
    row0 = sid * RPT
    nfull = RPT // K
    rem = RPT - nfull * K
    for q in range(nfull):
        pltpu.sync_copy(rows, acc.at[pl.ds(row0 + q * K, K)])
    if rem:
        pltpu.sync_copy(rows.at[pl.ds(0, rem)],
                        acc.at[pl.ds(row0 + nfull * K, rem)])
    plsc.subcore_barrier()

    g = gv[0, pl.ds(0, L)][0]
    lane = lax.broadcasted_iota(jnp.int32, (L,), 0)

    def _chunk(t, _):
        ci = w * cpw + t
        for j in range(SB):
            pltpu.sync_copy(src_hbm.at[ci, j], srcb[j])
            pltpu.sync_copy(dst_hbm.at[ci, j], dstb[j])
        descs = []
        for j in range(SB):
            descs.append(pltpu.async_copy(
                table_hbm.at[srcb[j]], rows.at[pl.ds(j * KB, KB)], gsem))
            descs.append(pltpu.async_copy(ed_hbm.at[dstb[j]], edb[j], gsem))
        for dsc in descs:
            dsc.wait()

        for j in range(SB):
            def _group(q, _, j=j):
                k0 = q * L
                kidx = k0 + lane
                ed_g = plsc.load_gather(edb[j],
                                        [kidx, jnp.zeros((L,), jnp.int32)])
                kv = (j * KB + k0) + lane
                es_g = plsc.load_gather(rows, [kv, jnp.full((L,), HID, jnp.int32)])
                e = _lrelu(es_g + ed_g)
                c = _lrelu(g + ed_g)
                f = jnp.exp(e - c)
                for i in range(L):
                    kk = j * KB + k0 + i
                    fs = f[i]
                    r0 = rows[kk, pl.ds(0, L)]
                    r1 = rows[kk, pl.ds(L, L)]
                    rows[kk, pl.ds(0, L)] = r0 * fs
                    rows[kk, pl.ds(L, L)] = jnp.where(lane == (HID - L), fs,
                                                      r1 * fs)
                return 0
            lax.fori_loop(0, KB // L, _group, 0)

        sdescs = []
        for j in range(SB):
            sdescs.append(pltpu.async_copy(
                rows.at[pl.ds(j * KB, KB)], acc.at[dstb[j]], ssem,
                add=True))
        for dsc in sdescs:
            dsc.wait()
        return 0

    lax.fori_loop(0, cpw, _chunk, 0)

    plsc.subcore_barrier()
    pltpu.sync_copy(acc.at[pl.ds(row0, RPT)],
                    out_hbm.at[cid, pl.ds(row0, RPT)])


def _edge_pass(src3, dst3, table, ed, gmax):
    cpw = src3.shape[0] // NW
    body = functools.partial(_edge_body, cpw)
    return pl.kernel(
        body,
        out_type=jax.ShapeDtypeStruct((NC, NP, TW), jnp.float32),
        mesh=plsc.VectorSubcoreMesh(core_axis_name="c", subcore_axis_name="s",
                                    num_cores=NC, num_subcores=NS),
        compiler_params=pltpu.CompilerParams(needs_layout_passes=False,
                                             use_tc_tiling_on_sc=False),
        scratch_types=[
            [pltpu.VMEM((KB,), jnp.int32)] * SB,
            [pltpu.VMEM((KB,), jnp.int32)] * SB,
            [pltpu.VMEM((KB, 8), jnp.float32)] * SB,
            pltpu.VMEM((K, TW), jnp.float32),
            pltpu.VMEM((1, 16), jnp.float32),
            pltpu.VMEM_SHARED((NP, TW), jnp.float32),
            pltpu.SemaphoreType.DMA,
            pltpu.SemaphoreType.DMA,
        ],
    )(src3, dst3, table, ed, gmax)


def _c_body(l, accp_ref, prev_ref, emb_ref, gam_ref, bet_ref, mu_ref, var_ref,
            b_ref, lw_ref, wn_ref, a2_ref, ow_ref, ob_ref,
            out_ref, embo_ref, table_ref, ed_ref, gmax_ref, fin_ref):
    i = pl.program_id(0)
    s = accp_ref[0] + accp_ref[1]
    agg = s[:, 0:HID]
    den = s[:, HID:HID + 1]
    y = agg / (den + 1e-16)
    scale = gam_ref[...] * lax.rsqrt(var_ref[...] + 1e-5)
    y = (y + b_ref[...] - mu_ref[...]) * scale + bet_ref[...]
    y = jnp.maximum(y, 0.0)
    if l > 0:
        y = y + 0.7 * prev_ref[...]
    out_ref[...] = y
    lw = lw_ref[...]
    wsm = jax.nn.softmax(lw, axis=1)
    emb = wsm[0, l] * y
    if l > 0:
        emb = emb + emb_ref[...]
    if l < 3:
        embo_ref[...] = emb
        h = jnp.dot(y, wn_ref[...], preferred_element_type=jnp.float32)
        esed = jnp.dot(h, a2_ref[...], preferred_element_type=jnp.float32)
        rows = i * R + lax.broadcasted_iota(jnp.int32, (R, 1), 0)
        valid = rows < N
        pad = jnp.zeros((R, TW - HID - 2), jnp.float32)
        table_ref[...] = jnp.concatenate([h, esed, pad], axis=1)
        edc = jnp.where(valid, esed[:, 1:2], 0.0)
        ed_ref[...] = jnp.concatenate([edc, jnp.zeros((R, 7), jnp.float32)],
                                      axis=1)
        es_m = jnp.where(valid, esed[:, 0:1], NEG)
        bm = jnp.max(es_m)

        @pl.when(i == 0)
        def _():
            gmax_ref[...] = jnp.full((1, 16), NEG, jnp.float32)

        gmax_ref[...] = jnp.maximum(gmax_ref[...], bm)
    else:
        fin_ref[...] = jnp.dot(emb, ow_ref[...],
                               preferred_element_type=jnp.float32) + ob_ref[...]


def _post(l, accp, prev, emb, bnp, bvec, lw, wn, a2, ow, ob):
    v20 = lambda a: a.reshape(1, HID)
    body = functools.partial(_c_body, l)
    small = lambda s: pl.BlockSpec(s, lambda i: (0, 0))
    outs = [
        jax.ShapeDtypeStruct((N, HID), jnp.float32),
        jax.ShapeDtypeStruct((N, HID), jnp.float32),
        jax.ShapeDtypeStruct((N, TW), jnp.float32),
        jax.ShapeDtypeStruct((NP, 8), jnp.float32),
        jax.ShapeDtypeStruct((1, 16), jnp.float32),
        jax.ShapeDtypeStruct((N, 2), jnp.float32),
    ]
    out_specs = [
        pl.BlockSpec((R, HID), lambda i: (i, 0)),
        pl.BlockSpec((R, HID), lambda i: (i, 0)),
        pl.BlockSpec((R, TW), lambda i: (i, 0)),
        pl.BlockSpec((R, 8), lambda i: (i, 0)),
        pl.BlockSpec((1, 16), lambda i: (0, 0)),
        pl.BlockSpec((R, 2), lambda i: (i, 0)),
    ]
    return pl.pallas_call(
        body,
        grid=(NS,),
        in_specs=[
            pl.BlockSpec((2, R, TW), lambda i: (0, i, 0)),
            pl.BlockSpec((R, HID), lambda i: (i, 0)),
            pl.BlockSpec((R, HID), lambda i: (i, 0)),
            small((1, HID)), small((1, HID)), small((1, HID)),
            small((1, HID)), small((1, HID)), small((1, 4)),
            small((HID, HID)), small((HID, 2)), small((HID, 2)),
            small((1, 2)),
        ],
        out_specs=out_specs,
        out_shape=outs,
    )(accp, prev, emb,
      v20(bnp['gamma']), v20(bnp['beta']), v20(bnp['mean']), v20(bnp['var']),
      bvec.reshape(1, HID), lw.reshape(1, 4), wn, a2, ow, ob.reshape(1, 2))


def kernel(features, edges, edges_weight, params):
    del edges_weight
    gat = params['gat']
    bn = params['bn']

    e_real = edges.shape[1] + N
    ct = -(-e_real // K)
    ct = -(-ct // NW) * NW
    ep = ct * K
    loop = jnp.arange(N, dtype=jnp.int32)
    srcp = jnp.concatenate(
        [edges[0], loop, jnp.zeros((ep - e_real,), jnp.int32)])
    dstp = jnp.concatenate(
        [edges[1], loop, jnp.full((ep - e_real,), N, jnp.int32)])
    src3 = srcp.reshape(ct, SB, KB)
    dst3 = dstp.reshape(ct, SB, KB)

    a2 = [jnp.stack([p['a_src'], p['a_dst']], axis=1) for p in gat]

    table, ed, gmax = _dense0(features, gat[0]['W'], a2[0])
    prev = jnp.zeros((N, HID), jnp.float32)
    emb = jnp.zeros((N, HID), jnp.float32)
    fin = None
    for l in range(4):
        accp = _edge_pass(src3, dst3, table, ed, gmax)
        wn = gat[l + 1]['W'] if l < 3 else jnp.zeros((HID, HID), jnp.float32)
        a2n = a2[l + 1] if l < 3 else jnp.zeros((HID, 2), jnp.float32)
        out_l, emb_n, table, ed, gmax, fin = _post(
            l, accp, prev, emb, bn[l], gat[l]['b'], params['layer_weights'],
            wn, a2n, params['out_W'], params['out_b'])
        prev, emb = out_l, emb_n
    return fin


# speedup vs baseline: 145.1326x; 1.7009x over previous
import functools

import jax
import jax.numpy as jnp
from jax import lax
from jax.experimental import pallas as pl
from jax.experimental.pallas import tpu as pltpu
import jax.experimental.pallas.tpu_sc as plsc

N = 50000
HID = 20
TW = 32
NC, NS, L = 2, 16, 16
NW = NC * NS
KB = 128
SB = 2
K = KB * SB
R = 3128
NP = R * NS
RPT = NP // NS
NEG = -1.0e30


def _lrelu(x):
    return jnp.where(x >= 0, x, 0.2 * x)


def _a_body(x_ref, w_ref, a2_ref, table_ref, ed_ref, gmax_ref):
    i = pl.program_id(0)
    h = jnp.dot(x_ref[...], w_ref[...], preferred_element_type=jnp.float32)
    esed = jnp.dot(h, a2_ref[...], preferred_element_type=jnp.float32)
    rows = i * R + lax.broadcasted_iota(jnp.int32, (R, 1), 0)
    valid = rows < N
    pad = jnp.zeros((R, TW - HID - 2), jnp.float32)
    table_ref[...] = jnp.concatenate([h, esed, pad], axis=1)
    edc = jnp.where(valid, esed[:, 1:2], 0.0)
    ed_ref[...] = jnp.concatenate([edc, jnp.zeros((R, 7), jnp.float32)],
                                  axis=1)
    es_m = jnp.where(valid, esed[:, 0:1], NEG)
    bm = jnp.max(es_m)

    @pl.when(i == 0)
    def _():
        gmax_ref[...] = jnp.full((1, 16), NEG, jnp.float32)

    gmax_ref[...] = jnp.maximum(gmax_ref[...], bm)


def _dense0(features, w0, a2):
    return pl.pallas_call(
        _a_body,
        grid=(NS,),
        in_specs=[
            pl.BlockSpec((R, features.shape[1]), lambda i: (i, 0)),
            pl.BlockSpec(w0.shape, lambda i: (0, 0)),
            pl.BlockSpec(a2.shape, lambda i: (0, 0)),
        ],
        out_specs=[
            pl.BlockSpec((R, TW), lambda i: (i, 0)),
            pl.BlockSpec((R, 8), lambda i: (i, 0)),
            pl.BlockSpec((1, 16), lambda i: (0, 0)),
        ],
        out_shape=[
            jax.ShapeDtypeStruct((N, TW), jnp.float32),
            jax.ShapeDtypeStruct((NP, 8), jnp.float32),
            jax.ShapeDtypeStruct((1, 16), jnp.float32),
        ],
    )(features, w0, a2)


def _edge_body(cpw, idx_hbm, table_hbm, ed_hbm, gmax_hbm, out_hbm,
               idxv, edb, rows, gv, acc, gsems, ssems):
    cid = lax.axis_index("c")
    sid = lax.axis_index("s")
    w = sid * NC + cid

    pltpu.sync_copy(gmax_hbm, gv)

    zv = jnp.zeros((L,), jnp.float32)

    def _zero_rows(i, _):
        rows[0][i, pl.ds(0, L)] = zv
        rows[0][i, pl.ds(L, L)] = zv
        return 0

    lax.fori_loop(0, K, _zero_rows, 0)
    row0 = sid * RPT
    nfull = RPT // K
    rem = RPT - nfull * K
    for q in range(nfull):
        pltpu.sync_copy(rows[0], acc.at[pl.ds(row0 + q * K, K)])
    if rem:
        pltpu.sync_copy(rows[0].at[pl.ds(0, rem)],
                        acc.at[pl.ds(row0 + nfull * K, rem)])
    plsc.subcore_barrier()

    g = gv[0, pl.ds(0, L)][0]
    lane = lax.broadcasted_iota(jnp.int32, (L,), 0)
    base = w * cpw

    def _gather_descs(b):
        ds_ = []
        for j in range(SB):
            ds_.append(pltpu.make_async_copy(
                table_hbm.at[idxv[b].at[0, j]], rows[b].at[pl.ds(j * KB, KB)],
                gsems[b]))
            ds_.append(pltpu.make_async_copy(
                ed_hbm.at[idxv[b].at[1, j]], edb[b][j], gsems[b]))
        return ds_

    def _scatter_descs(b):
        return [pltpu.make_async_copy(
            rows[b].at[pl.ds(j * KB, KB)], acc.at[idxv[b].at[1, j]],
            ssems[b]) for j in range(SB)]

    def _issue(c, b):
        pltpu.sync_copy(idx_hbm.at[c], idxv[b])
        for j in range(SB):
            pltpu.async_copy(
                table_hbm.at[idxv[b].at[0, j]], rows[b].at[pl.ds(j * KB, KB)],
                gsems[b])
            pltpu.async_copy(ed_hbm.at[idxv[b].at[1, j]], edb[b][j], gsems[b])

    def _scatter(b):
        for j in range(SB):
            pltpu.async_copy(rows[b].at[pl.ds(j * KB, KB)],
                             acc.at[idxv[b].at[1, j]], ssems[b], add=True)

    def _compute(b):
        for j in range(SB):
            def _group(q, _, j=j):
                k0 = q * L
                kidx = k0 + lane
                ed_g = plsc.load_gather(edb[b][j],
                                        [kidx, jnp.zeros((L,), jnp.int32)])
                kv = (j * KB + k0) + lane
                es_g = plsc.load_gather(
                    rows[b], [kv, jnp.full((L,), HID, jnp.int32)])
                e = _lrelu(es_g + ed_g)
                c = _lrelu(g + ed_g)
                f = jnp.exp(e - c)
                for i in range(L):
                    kk = j * KB + k0 + i
                    fs = f[i]
                    r0 = rows[b][kk, pl.ds(0, L)]
                    r1 = rows[b][kk, pl.ds(L, L)]
                    rows[b][kk, pl.ds(0, L)] = r0 * fs
                    rows[b][kk, pl.ds(L, L)] = jnp.where(
                        lane == (HID - L), fs, r1 * fs)
                return 0
            lax.fori_loop(0, KB // L, _group, 0)

    _issue(base, 0)

    def _pair(tt, _):
        for dsc in _gather_descs(0):
            dsc.wait()

        @pl.when(tt > 0)
        def _():
            for dsc in _scatter_descs(1):
                dsc.wait()

        _issue(base + 2 * tt + 1, 1)
        _compute(0)
        _scatter(0)
        for dsc in _gather_descs(1):
            dsc.wait()
        for dsc in _scatter_descs(0):
            dsc.wait()

        @pl.when(tt < cpw // 2 - 1)
        def _():
            _issue(base + 2 * tt + 2, 0)

        _compute(1)
        _scatter(1)
        return 0

    lax.fori_loop(0, cpw // 2, _pair, 0)
    for dsc in _scatter_descs(1):
        dsc.wait()

    plsc.subcore_barrier()
    pltpu.sync_copy(acc.at[pl.ds(row0, RPT)],
                    out_hbm.at[cid, pl.ds(row0, RPT)])


def _edge_pass(idx4, table, ed, gmax):
    cpw = idx4.shape[0] // NW
    body = functools.partial(_edge_body, cpw)
    return pl.kernel(
        body,
        out_type=jax.ShapeDtypeStruct((NC, NP, TW), jnp.float32),
        mesh=plsc.VectorSubcoreMesh(core_axis_name="c", subcore_axis_name="s",
                                    num_cores=NC, num_subcores=NS),
        compiler_params=pltpu.CompilerParams(needs_layout_passes=False,
                                             use_tc_tiling_on_sc=False),
        scratch_types=[
            [pltpu.VMEM((2, SB, KB), jnp.int32)] * 2,
            [[pltpu.VMEM((KB, 8), jnp.float32)] * SB] * 2,
            [pltpu.VMEM((K, TW), jnp.float32)] * 2,
            pltpu.VMEM((1, 16), jnp.float32),
            pltpu.VMEM_SHARED((NP, TW), jnp.float32),
            [pltpu.SemaphoreType.DMA] * 2,
            [pltpu.SemaphoreType.DMA] * 2,
        ],
    )(idx4, table, ed, gmax)


def _c_body(l, accp_ref, prev_ref, emb_ref, gam_ref, bet_ref, mu_ref, var_ref,
            b_ref, lw_ref, wn_ref, a2_ref, ow_ref, ob_ref,
            out_ref, embo_ref, table_ref, ed_ref, gmax_ref, fin_ref):
    i = pl.program_id(0)
    s = accp_ref[0] + accp_ref[1]
    agg = s[:, 0:HID]
    den = s[:, HID:HID + 1]
    y = agg / (den + 1e-16)
    scale = gam_ref[...] * lax.rsqrt(var_ref[...] + 1e-5)
    y = (y + b_ref[...] - mu_ref[...]) * scale + bet_ref[...]
    y = jnp.maximum(y, 0.0)
    if l > 0:
        y = y + 0.7 * prev_ref[...]
    out_ref[...] = y
    lw = lw_ref[...]
    wsm = jax.nn.softmax(lw, axis=1)
    emb = wsm[0, l] * y
    if l > 0:
        emb = emb + emb_ref[...]
    if l < 3:
        embo_ref[...] = emb
        h = jnp.dot(y, wn_ref[...], preferred_element_type=jnp.float32)
        esed = jnp.dot(h, a2_ref[...], preferred_element_type=jnp.float32)
        rows = i * R + lax.broadcasted_iota(jnp.int32, (R, 1), 0)
        valid = rows < N
        pad = jnp.zeros((R, TW - HID - 2), jnp.float32)
        table_ref[...] = jnp.concatenate([h, esed, pad], axis=1)
        edc = jnp.where(valid, esed[:, 1:2], 0.0)
        ed_ref[...] = jnp.concatenate([edc, jnp.zeros((R, 7), jnp.float32)],
                                      axis=1)
        es_m = jnp.where(valid, esed[:, 0:1], NEG)
        bm = jnp.max(es_m)

        @pl.when(i == 0)
        def _():
            gmax_ref[...] = jnp.full((1, 16), NEG, jnp.float32)

        gmax_ref[...] = jnp.maximum(gmax_ref[...], bm)
    else:
        fin_ref[...] = jnp.dot(emb, ow_ref[...],
                               preferred_element_type=jnp.float32) + ob_ref[...]


def _post(l, accp, prev, emb, bnp, bvec, lw, wn, a2, ow, ob):
    v20 = lambda a: a.reshape(1, HID)
    body = functools.partial(_c_body, l)
    small = lambda s: pl.BlockSpec(s, lambda i: (0, 0))
    outs = [
        jax.ShapeDtypeStruct((N, HID), jnp.float32),
        jax.ShapeDtypeStruct((N, HID), jnp.float32),
        jax.ShapeDtypeStruct((N, TW), jnp.float32),
        jax.ShapeDtypeStruct((NP, 8), jnp.float32),
        jax.ShapeDtypeStruct((1, 16), jnp.float32),
        jax.ShapeDtypeStruct((N, 2), jnp.float32),
    ]
    out_specs = [
        pl.BlockSpec((R, HID), lambda i: (i, 0)),
        pl.BlockSpec((R, HID), lambda i: (i, 0)),
        pl.BlockSpec((R, TW), lambda i: (i, 0)),
        pl.BlockSpec((R, 8), lambda i: (i, 0)),
        pl.BlockSpec((1, 16), lambda i: (0, 0)),
        pl.BlockSpec((R, 2), lambda i: (i, 0)),
    ]
    return pl.pallas_call(
        body,
        grid=(NS,),
        in_specs=[
            pl.BlockSpec((2, R, TW), lambda i: (0, i, 0)),
            pl.BlockSpec((R, HID), lambda i: (i, 0)),
            pl.BlockSpec((R, HID), lambda i: (i, 0)),
            small((1, HID)), small((1, HID)), small((1, HID)),
            small((1, HID)), small((1, HID)), small((1, 4)),
            small((HID, HID)), small((HID, 2)), small((HID, 2)),
            small((1, 2)),
        ],
        out_specs=out_specs,
        out_shape=outs,
    )(accp, prev, emb,
      v20(bnp['gamma']), v20(bnp['beta']), v20(bnp['mean']), v20(bnp['var']),
      bvec.reshape(1, HID), lw.reshape(1, 4), wn, a2, ow, ob.reshape(1, 2))


def kernel(features, edges, edges_weight, params):
    del edges_weight
    gat = params['gat']
    bn = params['bn']

    e_real = edges.shape[1] + N
    ct = -(-e_real // K)
    ct = -(-ct // (2 * NW)) * (2 * NW)
    ep = ct * K
    loop = jnp.arange(N, dtype=jnp.int32)
    srcp = jnp.concatenate(
        [edges[0], loop, jnp.zeros((ep - e_real,), jnp.int32)])
    dstp = jnp.concatenate(
        [edges[1], loop, jnp.full((ep - e_real,), N, jnp.int32)])
    idx4 = jnp.stack([srcp.reshape(ct, SB, KB), dstp.reshape(ct, SB, KB)],
                     axis=1)

    a2 = [jnp.stack([p['a_src'], p['a_dst']], axis=1) for p in gat]

    table, ed, gmax = _dense0(features, gat[0]['W'], a2[0])
    prev = jnp.zeros((N, HID), jnp.float32)
    emb = jnp.zeros((N, HID), jnp.float32)
    fin = None
    for l in range(4):
        accp = _edge_pass(idx4, table, ed, gmax)
        wn = gat[l + 1]['W'] if l < 3 else jnp.zeros((HID, HID), jnp.float32)
        a2n = a2[l + 1] if l < 3 else jnp.zeros((HID, 2), jnp.float32)
        out_l, emb_n, table, ed, gmax, fin = _post(
            l, accp, prev, emb, bn[l], gat[l]['b'], params['layer_weights'],
            wn, a2n, params['out_W'], params['out_b'])
        prev, emb = out_l, emb_n
    return fin


# speedup vs baseline: 145.1975x; 1.0004x over previous
import functools

import jax
import jax.numpy as jnp
from jax import lax
from jax.experimental import pallas as pl
from jax.experimental.pallas import tpu as pltpu
import jax.experimental.pallas.tpu_sc as plsc

N = 50000
HID = 20
TW = 32
NC, NS, L = 2, 16, 16
NW = NC * NS
KB = 128
SB = 2
K = KB * SB
AW = 32
R = 3128
NP = R * NS
RPT = NP // NS
NEG = -1.0e30


def _lrelu(x):
    return jnp.where(x >= 0, x, 0.2 * x)


def _a_body(x_ref, w_ref, a2_ref, table_ref, ed_ref, gmax_ref):
    i = pl.program_id(0)
    h = jnp.dot(x_ref[...], w_ref[...], preferred_element_type=jnp.float32)
    esed = jnp.dot(h, a2_ref[...], preferred_element_type=jnp.float32)
    rows = i * R + lax.broadcasted_iota(jnp.int32, (R, 1), 0)
    valid = rows < N
    pad = jnp.zeros((R, TW - HID - 2), jnp.float32)
    table_ref[...] = jnp.concatenate([h, esed, pad], axis=1)
    edc = jnp.where(valid, esed[:, 1:2], 0.0)
    ed_ref[...] = jnp.concatenate([edc, jnp.zeros((R, 7), jnp.float32)],
                                  axis=1)
    es_m = jnp.where(valid, esed[:, 0:1], NEG)
    bm = jnp.max(es_m)

    @pl.when(i == 0)
    def _():
        gmax_ref[...] = jnp.full((1, 16), NEG, jnp.float32)

    gmax_ref[...] = jnp.maximum(gmax_ref[...], bm)


def _dense0(features, w0, a2):
    return pl.pallas_call(
        _a_body,
        grid=(NS,),
        in_specs=[
            pl.BlockSpec((R, features.shape[1]), lambda i: (i, 0)),
            pl.BlockSpec(w0.shape, lambda i: (0, 0)),
            pl.BlockSpec(a2.shape, lambda i: (0, 0)),
        ],
        out_specs=[
            pl.BlockSpec((R, TW), lambda i: (i, 0)),
            pl.BlockSpec((R, 8), lambda i: (i, 0)),
            pl.BlockSpec((1, 16), lambda i: (0, 0)),
        ],
        out_shape=[
            jax.ShapeDtypeStruct((N, TW), jnp.float32),
            jax.ShapeDtypeStruct((NP, 8), jnp.float32),
            jax.ShapeDtypeStruct((1, 16), jnp.float32),
        ],
    )(features, w0, a2)


def _edge_body(cpw, idx_hbm, table_hbm, ed_hbm, gmax_hbm, out_hbm,
               idxv, edb, rows, gv, acc, gsems, ssems):
    cid = lax.axis_index("c")
    sid = lax.axis_index("s")
    w = sid * NC + cid

    pltpu.sync_copy(gmax_hbm, gv)

    zv = jnp.zeros((L,), jnp.float32)

    def _zero_rows(i, _):
        rows[0][i, pl.ds(0, L)] = zv
        rows[0][i, pl.ds(TW - L, L)] = zv
        return 0

    lax.fori_loop(0, K, _zero_rows, 0)
    row0 = sid * RPT
    nfull = RPT // K
    rem = RPT - nfull * K
    for q in range(nfull):
        pltpu.sync_copy(rows[0], acc.at[pl.ds(row0 + q * K, K)])
    if rem:
        pltpu.sync_copy(rows[0].at[pl.ds(0, rem)],
                        acc.at[pl.ds(row0 + nfull * K, rem)])
    plsc.subcore_barrier()

    g = gv[0, pl.ds(0, L)][0]
    lane = lax.broadcasted_iota(jnp.int32, (L,), 0)
    base = w * cpw

    def _gather_descs(b):
        ds_ = []
        for j in range(SB):
            ds_.append(pltpu.make_async_copy(
                table_hbm.at[idxv[b].at[0, j]], rows[b].at[pl.ds(j * KB, KB)],
                gsems[b]))
            ds_.append(pltpu.make_async_copy(
                ed_hbm.at[idxv[b].at[1, j]], edb[b][j], gsems[b]))
        return ds_

    def _scatter_descs(b):
        return [pltpu.make_async_copy(
            rows[b].at[pl.ds(j * KB, KB)], acc.at[idxv[b].at[1, j]],
            ssems[b]) for j in range(SB)]

    def _issue(c, b):
        pltpu.sync_copy(idx_hbm.at[c], idxv[b])
        for j in range(SB):
            pltpu.async_copy(
                table_hbm.at[idxv[b].at[0, j]], rows[b].at[pl.ds(j * KB, KB)],
                gsems[b])
            pltpu.async_copy(ed_hbm.at[idxv[b].at[1, j]], edb[b][j], gsems[b])

    def _scatter(b):
        for j in range(SB):
            pltpu.async_copy(rows[b].at[pl.ds(j * KB, KB)],
                             acc.at[idxv[b].at[1, j]], ssems[b], add=True)

    def _compute(b):
        for j in range(SB):
            def _group(q, _, j=j):
                k0 = q * L
                kidx = k0 + lane
                ed_g = plsc.load_gather(edb[b][j],
                                        [kidx, jnp.zeros((L,), jnp.int32)])
                kv = (j * KB + k0) + lane
                es_g = plsc.load_gather(
                    rows[b], [kv, jnp.full((L,), HID, jnp.int32)])
                e = _lrelu(es_g + ed_g)
                c = _lrelu(g + ed_g)
                f = jnp.exp(e - c)
                for i in range(L):
                    kk = j * KB + k0 + i
                    fs = f[i]
                    r0 = rows[b][kk, pl.ds(0, L)]
                    rows[b][kk, pl.ds(0, L)] = r0 * fs
                    r1 = rows[b][kk, pl.ds(TW - L, L)]
                    s1 = jnp.where(lane < 2 * L - TW, r1, r1 * fs)
                    s1 = jnp.where(lane == HID - (TW - L), fs, s1)
                    rows[b][kk, pl.ds(TW - L, L)] = s1
                return 0
            lax.fori_loop(0, KB // L, _group, 0)

    _issue(base, 0)

    def _pair(tt, _):
        for dsc in _gather_descs(0):
            dsc.wait()

        @pl.when(tt > 0)
        def _():
            for dsc in _scatter_descs(1):
                dsc.wait()

        _issue(base + 2 * tt + 1, 1)
        _compute(0)
        _scatter(0)
        for dsc in _gather_descs(1):
            dsc.wait()
        for dsc in _scatter_descs(0):
            dsc.wait()

        @pl.when(tt < cpw // 2 - 1)
        def _():
            _issue(base + 2 * tt + 2, 0)

        _compute(1)
        _scatter(1)
        return 0

    lax.fori_loop(0, cpw // 2, _pair, 0)
    for dsc in _scatter_descs(1):
        dsc.wait()

    plsc.subcore_barrier()
    pltpu.sync_copy(acc.at[pl.ds(row0, RPT)],
                    out_hbm.at[cid, pl.ds(row0, RPT)])


def _edge_pass(idx4, table, ed, gmax):
    cpw = idx4.shape[0] // NW
    body = functools.partial(_edge_body, cpw)
    return pl.kernel(
        body,
        out_type=jax.ShapeDtypeStruct((NC, NP, AW), jnp.float32),
        mesh=plsc.VectorSubcoreMesh(core_axis_name="c", subcore_axis_name="s",
                                    num_cores=NC, num_subcores=NS),
        compiler_params=pltpu.CompilerParams(needs_layout_passes=False,
                                             use_tc_tiling_on_sc=False),
        scratch_types=[
            [pltpu.VMEM((2, SB, KB), jnp.int32)] * 2,
            [[pltpu.VMEM((KB, 8), jnp.float32)] * SB] * 2,
            [pltpu.VMEM((K, TW), jnp.float32)] * 2,
            pltpu.VMEM((1, 16), jnp.float32),
            pltpu.VMEM_SHARED((NP, AW), jnp.float32),
            [pltpu.SemaphoreType.DMA] * 2,
            [pltpu.SemaphoreType.DMA] * 2,
        ],
    )(idx4, table, ed, gmax)


def _c_body(l, accp_ref, prev_ref, emb_ref, gam_ref, bet_ref, mu_ref, var_ref,
            b_ref, lw_ref, wn_ref, a2_ref, ow_ref, ob_ref,
            out_ref, embo_ref, table_ref, ed_ref, gmax_ref, fin_ref):
    i = pl.program_id(0)
    s = accp_ref[0] + accp_ref[1]
    agg = s[:, 0:HID]
    den = s[:, HID:HID + 1]
    y = agg / (den + 1e-16)
    scale = gam_ref[...] * lax.rsqrt(var_ref[...] + 1e-5)
    y = (y + b_ref[...] - mu_ref[...]) * scale + bet_ref[...]
    y = jnp.maximum(y, 0.0)
    if l > 0:
        y = y + 0.7 * prev_ref[...]
    out_ref[...] = y
    lw = lw_ref[...]
    wsm = jax.nn.softmax(lw, axis=1)
    emb = wsm[0, l] * y
    if l > 0:
        emb = emb + emb_ref[...]
    if l < 3:
        embo_ref[...] = emb
        h = jnp.dot(y, wn_ref[...], preferred_element_type=jnp.float32)
        esed = jnp.dot(h, a2_ref[...], preferred_element_type=jnp.float32)
        rows = i * R + lax.broadcasted_iota(jnp.int32, (R, 1), 0)
        valid = rows < N
        pad = jnp.zeros((R, TW - HID - 2), jnp.float32)
        table_ref[...] = jnp.concatenate([h, esed, pad], axis=1)
        edc = jnp.where(valid, esed[:, 1:2], 0.0)
        ed_ref[...] = jnp.concatenate([edc, jnp.zeros((R, 7), jnp.float32)],
                                      axis=1)
        es_m = jnp.where(valid, esed[:, 0:1], NEG)
        bm = jnp.max(es_m)

        @pl.when(i == 0)
        def _():
            gmax_ref[...] = jnp.full((1, 16), NEG, jnp.float32)

        gmax_ref[...] = jnp.maximum(gmax_ref[...], bm)
    else:
        fin_ref[...] = jnp.dot(emb, ow_ref[...],
                               preferred_element_type=jnp.float32) + ob_ref[...]


def _post(l, accp, prev, emb, bnp, bvec, lw, wn, a2, ow, ob):
    v20 = lambda a: a.reshape(1, HID)
    body = functools.partial(_c_body, l)
    small = lambda s: pl.BlockSpec(s, lambda i: (0, 0))
    outs = [
        jax.ShapeDtypeStruct((N, HID), jnp.float32),
        jax.ShapeDtypeStruct((N, HID), jnp.float32),
        jax.ShapeDtypeStruct((N, TW), jnp.float32),
        jax.ShapeDtypeStruct((NP, 8), jnp.float32),
        jax.ShapeDtypeStruct((1, 16), jnp.float32),
        jax.ShapeDtypeStruct((N, 2), jnp.float32),
    ]
    out_specs = [
        pl.BlockSpec((R, HID), lambda i: (i, 0)),
        pl.BlockSpec((R, HID), lambda i: (i, 0)),
        pl.BlockSpec((R, TW), lambda i: (i, 0)),
        pl.BlockSpec((R, 8), lambda i: (i, 0)),
        pl.BlockSpec((1, 16), lambda i: (0, 0)),
        pl.BlockSpec((R, 2), lambda i: (i, 0)),
    ]
    return pl.pallas_call(
        body,
        grid=(NS,),
        in_specs=[
            pl.BlockSpec((2, R, AW), lambda i: (0, i, 0)),
            pl.BlockSpec((R, HID), lambda i: (i, 0)),
            pl.BlockSpec((R, HID), lambda i: (i, 0)),
            small((1, HID)), small((1, HID)), small((1, HID)),
            small((1, HID)), small((1, HID)), small((1, 4)),
            small((HID, HID)), small((HID, 2)), small((HID, 2)),
            small((1, 2)),
        ],
        out_specs=out_specs,
        out_shape=outs,
    )(accp, prev, emb,
      v20(bnp['gamma']), v20(bnp['beta']), v20(bnp['mean']), v20(bnp['var']),
      bvec.reshape(1, HID), lw.reshape(1, 4), wn, a2, ow, ob.reshape(1, 2))


def kernel(features, edges, edges_weight, params):
    del edges_weight
    gat = params['gat']
    bn = params['bn']

    e_real = edges.shape[1] + N
    ct = -(-e_real // K)
    ct = -(-ct // (2 * NW)) * (2 * NW)
    ep = ct * K
    loop = jnp.arange(N, dtype=jnp.int32)
    srcp = jnp.concatenate(
        [edges[0], loop, jnp.zeros((ep - e_real,), jnp.int32)])
    dstp = jnp.concatenate(
        [edges[1], loop, jnp.full((ep - e_real,), N, jnp.int32)])
    idx4 = jnp.stack([srcp.reshape(ct, SB, KB), dstp.reshape(ct, SB, KB)],
                     axis=1)

    a2 = [jnp.stack([p['a_src'], p['a_dst']], axis=1) for p in gat]

    table, ed, gmax = _dense0(features, gat[0]['W'], a2[0])
    prev = jnp.zeros((N, HID), jnp.float32)
    emb = jnp.zeros((N, HID), jnp.float32)
    fin = None
    for l in range(4):
        accp = _edge_pass(idx4, table, ed, gmax)
        wn = gat[l + 1]['W'] if l < 3 else jnp.zeros((HID, HID), jnp.float32)
        a2n = a2[l + 1] if l < 3 else jnp.zeros((HID, 2), jnp.float32)
        out_l, emb_n, table, ed, gmax, fin = _post(
            l, accp, prev, emb, bn[l], gat[l]['b'], params['layer_weights'],
            wn, a2n, params['out_W'], params['out_b'])
        prev, emb = out_l, emb_n
    return fin
